# baseline (speedup 1.0000x reference)
import functools
import math

import jax
import jax.numpy as jnp
from jax import lax
from jax.experimental import pallas as pl
from jax.experimental.pallas import tpu as pltpu
from jax.experimental.pallas import tpu_sc as plsc

_DIM = 64
_SCALE = math.sqrt(float(_DIM))

_NC = 2
_NS = 16
_NW = _NC * _NS

_IDX_ROW = 128
_CHUNK_ROWS = 4
_CHUNK = _IDX_ROW * _CHUNK_ROWS


@functools.lru_cache(maxsize=None)
def _build(n_idx_rows):
  rows_per_w = n_idx_rows // _NW
  chunks = rows_per_w // _CHUNK_ROWS
  b_total = n_idx_rows * _IDX_ROW
  mesh = plsc.VectorSubcoreMesh(core_axis_name="c", subcore_axis_name="s")

  @functools.partial(
      pl.kernel,
      mesh=mesh,
      out_type=jax.ShapeDtypeStruct((b_total, _DIM), jnp.float32),
      scratch_types=[
          pltpu.VMEM((_CHUNK_ROWS, _IDX_ROW), jnp.int32),
          pltpu.VMEM((_CHUNK, _DIM), jnp.float32),
          pltpu.SemaphoreType.DMA,
      ],
      compiler_params=pltpu.CompilerParams(use_tc_tiling_on_sc=False),
  )
  def k(idx_hbm, table_hbm, out_hbm, idx_v, rows_v, sem):
    wid = lax.axis_index("s") * _NC + lax.axis_index("c")
    row_base = wid * rows_per_w

    @pl.loop(0, chunks)
    def chunk_loop(g):
      rb = row_base + g * _CHUNK_ROWS
      pltpu.sync_copy(idx_hbm.at[pl.ds(rb, _CHUNK_ROWS)], idx_v)
      copies = [
          pltpu.async_copy(
              table_hbm.at[idx_v.at[j]],
              rows_v.at[pl.ds(j * _IDX_ROW, _IDX_ROW)],
              sem,
          )
          for j in range(_CHUNK_ROWS)
      ]
      for c in copies:
        c.wait()

      @plsc.parallel_loop(0, _CHUNK, unroll=8)
      def scale_loop(r):
        for c in range(_DIM // 16):
          sl = pl.ds(c * 16, 16)
          rows_v[r, sl] = rows_v[r, sl] * _SCALE

      pltpu.sync_copy(rows_v, out_hbm.at[pl.ds(rb * _IDX_ROW, _CHUNK)])

  return k


def kernel(inputs, table):
  b, t = inputs.shape
  n = b * t
  idx2d = inputs.reshape(n // _IDX_ROW, _IDX_ROW).astype(jnp.int32)
  out = _build(n // _IDX_ROW)(idx2d, table)
  return out.reshape(b, t, _DIM)


# speedup vs baseline: 1.0932x; 1.0932x over previous
import functools
import math

import jax
import jax.numpy as jnp
from jax import lax
from jax.experimental import pallas as pl
from jax.experimental.pallas import tpu as pltpu
from jax.experimental.pallas import tpu_sc as plsc

_DIM = 64
_SCALE = math.sqrt(float(_DIM))

_NC = 2
_NS = 16
_NW = _NC * _NS

_IDX_ROW = 128
_CHUNK_ROWS = 2
_CHUNK = _IDX_ROW * _CHUNK_ROWS
_NBUF = 4


@functools.lru_cache(maxsize=None)
def _build(n_idx_rows):
  rows_per_w = n_idx_rows // _NW
  chunks = rows_per_w // _CHUNK_ROWS
  outer = chunks // _NBUF
  assert rows_per_w * _NW == n_idx_rows
  assert chunks * _CHUNK_ROWS == rows_per_w
  assert outer * _NBUF == chunks and outer >= 3
  b_total = n_idx_rows * _IDX_ROW
  mesh = plsc.VectorSubcoreMesh(core_axis_name="c", subcore_axis_name="s")

  @functools.partial(
      pl.kernel,
      mesh=mesh,
      out_type=jax.ShapeDtypeStruct((b_total, _DIM), jnp.float32),
      scratch_types=(
          [pltpu.VMEM((_CHUNK_ROWS, _IDX_ROW), jnp.int32)] * _NBUF
          + [pltpu.VMEM((_CHUNK, _DIM), jnp.float32)] * _NBUF
          + [pltpu.SemaphoreType.DMA] * (3 * _NBUF)
      ),
      compiler_params=pltpu.CompilerParams(use_tc_tiling_on_sc=False),
  )
  def k(idx_hbm, table_hbm, out_hbm, *bufs):
    ibuf = bufs[:_NBUF]
    rbuf = bufs[_NBUF:2 * _NBUF]
    gsem = bufs[2 * _NBUF:3 * _NBUF]
    osem = bufs[3 * _NBUF:4 * _NBUF]
    isem = bufs[4 * _NBUF:5 * _NBUF]

    wid = lax.axis_index("s") * _NC + lax.axis_index("c")
    row_base = wid * rows_per_w

    def idx_row(g):
      return row_base + g * _CHUNK_ROWS

    def fire_gathers(g, b):
      del g
      for j in range(_CHUNK_ROWS):
        pltpu.async_copy(
            table_hbm.at[ibuf[b].at[j]],
            rbuf[b].at[pl.ds(j * _IDX_ROW, _IDX_ROW)],
            gsem[b],
        )

    def wait_gathers(b):
      for j in range(_CHUNK_ROWS):
        pltpu.make_async_copy(
            table_hbm.at[pl.ds(0, _IDX_ROW)],
            rbuf[b].at[pl.ds(j * _IDX_ROW, _IDX_ROW)],
            gsem[b],
        ).wait()

    def fire_idx(g, b):
      pltpu.async_copy(
          idx_hbm.at[pl.ds(idx_row(g), _CHUNK_ROWS)], ibuf[b], isem[b])

    def wait_idx(b):
      pltpu.make_async_copy(
          idx_hbm.at[pl.ds(0, _CHUNK_ROWS)], ibuf[b], isem[b]).wait()

    def fire_out(g, b):
      pltpu.async_copy(
          rbuf[b], out_hbm.at[pl.ds(idx_row(g) * _IDX_ROW, _CHUNK)], osem[b])

    def wait_out(b):
      pltpu.make_async_copy(
          rbuf[b], out_hbm.at[pl.ds(0, _CHUNK)], osem[b]).wait()

    def scale(b):
      ref = rbuf[b]

      @plsc.parallel_loop(0, _CHUNK, unroll=8)
      def _(r):
        for c in range(_DIM // 16):
          sl = pl.ds(c * 16, 16)
          ref[r, sl] = ref[r, sl] * _SCALE

    def body(g, b, first=False, fire_i=True, fire_g=True):
      wait_gathers(b)
      if fire_i:
        fire_idx(g + _NBUF, b)
      scale(b)
      fire_out(g, b)
      if fire_g:
        bn = (b + _NBUF - 1) % _NBUF
        if not first:
          wait_out(bn)
        wait_idx(bn)
        fire_gathers(g + _NBUF - 1, bn)

    for b in range(_NBUF - 1):
      pltpu.sync_copy(idx_hbm.at[pl.ds(idx_row(b), _CHUNK_ROWS)], ibuf[b])
      fire_gathers(b, b)
    fire_idx(_NBUF - 1, _NBUF - 1)

    body(0, 0, first=True)
    for b in range(1, _NBUF):
      body(b, b)

    @pl.loop(1, outer - 1)
    def _(i):
      g0 = i * _NBUF
      for b in range(_NBUF):
        body(g0 + b, b)

    gl = (outer - 1) * _NBUF
    body(gl, 0, fire_i=False)
    for b in range(1, _NBUF):
      body(gl + b, b, fire_i=False, fire_g=False)

    for b in range(_NBUF):
      wait_out(b)

  return k


def kernel(inputs, table):
  b, t = inputs.shape
  n = b * t
  idx2d = inputs.reshape(n // _IDX_ROW, _IDX_ROW).astype(jnp.int32)
  out = _build(n // _IDX_ROW)(idx2d, table)
  return out.reshape(b, t, _DIM)
